# baseline (speedup 1.0000x reference)
import functools

import jax
import jax.numpy as jnp
from jax import lax
from jax.experimental import pallas as pl
from jax.experimental.pallas import tpu as pltpu
from jax.experimental.pallas import tpu_sc as plsc

NC = 2
NS = 16
NW = NC * NS

B, T = 4096, 200
D = 64
N = B * T
PER_W = N // NW
G = 128
STEPS = PER_W // G


@functools.partial(
    pl.kernel,
    out_type=jax.ShapeDtypeStruct((N, D), jnp.float32),
    mesh=plsc.VectorSubcoreMesh(core_axis_name="c", subcore_axis_name="s"),
    scratch_types=[
        pltpu.VMEM((STEPS, G), jnp.int32),
        pltpu.VMEM((G, D), jnp.float32),
        pltpu.SemaphoreType.DMA,
    ],
    compiler_params=pltpu.CompilerParams(use_tc_tiling_on_sc=False),
)
def _gather_kernel(idx_hbm, table_hbm, out_hbm, idx_v, rows_v, sem):
    wid = lax.axis_index("s") * NC + lax.axis_index("c")
    base = wid * PER_W
    pltpu.sync_copy(idx_hbm.at[wid], idx_v)

    def step(i, carry):
        pltpu.async_copy(table_hbm.at[idx_v.at[i]], rows_v, sem).wait()
        pltpu.sync_copy(rows_v, out_hbm.at[pl.ds(base + i * G, G)])
        return carry

    lax.fori_loop(0, STEPS, step, 0)


def kernel(integers, table):
    idx = integers.reshape(NW, STEPS, G)
    out = _gather_kernel(idx, table)
    return out.reshape(B, T, D)


# speedup vs baseline: 1.1163x; 1.1163x over previous
import functools

import jax
import jax.numpy as jnp
from jax import lax
from jax.experimental import pallas as pl
from jax.experimental.pallas import tpu as pltpu
from jax.experimental.pallas import tpu_sc as plsc

NC = 2
NS = 16
NW = NC * NS

B, T = 4096, 200
D = 64
N = B * T
PER_W = N // NW
G = 128
STEPS = PER_W // G
GPC = 4
W = G * GPC
CH = STEPS // GPC
NBUF = 3


@functools.partial(
    pl.kernel,
    out_type=jax.ShapeDtypeStruct((N, D), jnp.float32),
    mesh=plsc.VectorSubcoreMesh(core_axis_name="c", subcore_axis_name="s"),
    scratch_types=[
        pltpu.VMEM((STEPS, G), jnp.int32),
        pltpu.VMEM((NBUF, W, D), jnp.float32),
        pltpu.SemaphoreType.DMA,
        pltpu.SemaphoreType.DMA,
    ],
    compiler_params=pltpu.CompilerParams(use_tc_tiling_on_sc=False),
)
def _gather_kernel(idx_hbm, table_hbm, out_hbm, idx_v, bufs, gsem, wsem):
    wid = lax.axis_index("s") * NC + lax.axis_index("c")
    base = wid * PER_W
    pltpu.sync_copy(idx_hbm.at[wid], idx_v)

    def issue_chunk(j, b):
        for g in range(GPC):
            pltpu.async_copy(
                table_hbm.at[idx_v.at[j * GPC + g]],
                bufs.at[b, pl.ds(g * G, G)],
                gsem,
            )

    issue_chunk(0, 0)

    def body(j, carry):
        b = lax.rem(j, NBUF)
        bn = lax.rem(j + 1, NBUF)

        @pl.when(j >= 2)
        def _():
            pltpu.make_async_copy(
                bufs.at[bn], out_hbm.at[pl.ds(base, W)], wsem
            ).wait()

        @pl.when(j + 1 < CH)
        def _():
            issue_chunk(j + 1, bn)

        for g in range(GPC):
            pltpu.make_async_copy(
                table_hbm.at[idx_v.at[j * GPC + g]],
                bufs.at[b, pl.ds(g * G, G)],
                gsem,
            ).wait()

        pltpu.async_copy(bufs.at[b], out_hbm.at[pl.ds(base + j * W, W)], wsem)
        return carry

    lax.fori_loop(0, CH, body, 0)

    pltpu.make_async_copy(bufs.at[0], out_hbm.at[pl.ds(base, W)], wsem).wait()
    pltpu.make_async_copy(bufs.at[0], out_hbm.at[pl.ds(base, W)], wsem).wait()


def kernel(integers, table):
    idx = integers.reshape(NW, STEPS, G)
    out = _gather_kernel(idx, table)
    return out.reshape(B, T, D)
